# baseline (speedup 1.0000x reference)
import functools

import jax
import jax.numpy as jnp
from jax import lax
from jax.experimental import pallas as pl
from jax.experimental.pallas import tpu as pltpu
from jax.experimental.pallas import tpu_sc as plsc

NC = 2
NSUB = 16
NW = NC * NSUB
NB = 4
CF = 8
CV = 8


def _wsum(gref, rows, col, wts):
    acc = wts[0] * gref[rows[0], pl.ds(col, 16)]
    for t in range(1, len(wts)):
        acc = acc + wts[t] * gref[rows[t], pl.ds(col, 16)]
    return acc


def _face_lap_kernel(nf, nvp, cdim, x_hbm, idxf_hbm, wen_hbm, idxl_hbm,
                     wl_hbm, face_hbm, lap_hbm,
                     idxq0, idxq1, wbuf, wlbuf,
                     g0, g1, g2, g3, fo0, fo1, fo2, fo3, lo0, lo1, lo2, lo3,
                     sg0, sg1, sg2, sg3,
                     sf0, sf1, sf2, sf3, sl0, sl1, sl2, sl3):
    wid = lax.axis_index("s") * NC + lax.axis_index("c")
    ngrp = cdim // 16
    gbufs = (g0, g1, g2, g3)
    fobufs = (fo0, fo1, fo2, fo3)
    lobufs = (lo0, lo1, lo2, lo3)
    gsems = (sg0, sg1, sg2, sg3)
    fsems = (sf0, sf1, sf2, sf3)
    lsems = (sl0, sl1, sl2, sl3)
    idxqs = (idxq0, idxq1)

    fpw = nf // NW
    nch = fpw // CF
    cbase = wid * nch
    nper = CF * 9

    def fgather(b, idxq):
        pltpu.async_copy(x_hbm.at[idxq.at[pl.ds(b * nper, nper)]],
                         gbufs[b], gsems[b])

    def fwait(b):
        pltpu.make_async_copy(x_hbm.at[idxq0.at[pl.ds(0, nper)]],
                              gbufs[b], gsems[b]).wait()

    def fcompute(b, fb):
        gb, ob = gbufs[b], fobufs[b]
        def ibody(i, carry, gb=gb, ob=ob):
            wev = wbuf[pl.ds(i * 32, 16)]
            wnv = wbuf[pl.ds(i * 32 + 16, 16)]
            we = [wev[t] for t in range(9)]
            wn = [wnv[t] for t in range(9)]
            rows = [i * 9 + t for t in range(9)]
            def jbody(j, carry2, we=we, wn=wn, rows=rows, gb=gb, ob=ob, i=i):
                col = j * 16
                ob[i, pl.ds(col, 16)] = _wsum(gb, rows, col, we)
                ob[i, pl.ds(cdim + col, 16)] = _wsum(gb, rows, col, wn)
                return carry2
            lax.fori_loop(0, ngrp, jbody, 0, unroll=4)
            return carry
        lax.fori_loop(0, CF, ibody, 0)
        pltpu.async_copy(ob, face_hbm.at[pl.ds(b * nf + fb, CF)], fsems[b])

    def fdrain(b, fb):
        pltpu.make_async_copy(fobufs[b], face_hbm.at[pl.ds(b * nf + fb, CF)],
                              fsems[b]).wait()

    pltpu.sync_copy(idxf_hbm.at[pl.ds(cbase * NB * nper, NB * nper)], idxq0)
    for b in range(NB):
        fgather(b, idxq0)

    def face_body(c2, carry):
        for u in range(2):
            cc = c2 * 2 + u
            fb = wid * fpw + cc * CF
            ccn = jnp.minimum(cc + 1, nch - 1)
            pltpu.sync_copy(
                idxf_hbm.at[pl.ds((cbase + ccn) * NB * nper, NB * nper)],
                idxqs[1 - u])
            pltpu.sync_copy(wen_hbm.at[pl.ds((cbase + cc) * CF * 32,
                                             CF * 32)], wbuf)
            for b in range(NB):
                fwait(b)
                if u == 0:
                    @pl.when(c2 > 0)
                    def _(b=b, fb=fb):
                        fdrain(b, fb)
                else:
                    fdrain(b, fb)
                fcompute(b, fb)
                fgather(b, idxqs[1 - u])
        return carry

    lax.fori_loop(0, nch // 2, face_body, 0)
    last_fb = wid * fpw + (nch - 1) * CF
    for b in range(NB):
        fwait(b)
        fdrain(b, last_fb)

    vpw = nvp // NW
    nchl = vpw // CV
    lbase = wid * nchl
    lper = CV * 7

    def lgather(b, idxq):
        pltpu.async_copy(x_hbm.at[idxq.at[pl.ds(b * lper, lper)]],
                         gbufs[b].at[pl.ds(0, lper)], gsems[b])

    def lwait(b):
        pltpu.make_async_copy(x_hbm.at[idxq0.at[pl.ds(0, lper)]],
                              gbufs[b].at[pl.ds(0, lper)], gsems[b]).wait()

    def lcompute(b, vb):
        gb, ob = gbufs[b], lobufs[b]
        def ibody(i, carry, gb=gb, ob=ob):
            wlv = wlbuf[pl.ds(i * 8, 16)]
            wl = [wlv[t] for t in range(7)]
            rows = [i * 7 + t for t in range(7)]
            def jbody(j, carry2, wl=wl, rows=rows, gb=gb, ob=ob, i=i):
                col = j * 16
                ob[i, pl.ds(col, 16)] = _wsum(gb, rows, col, wl)
                return carry2
            lax.fori_loop(0, ngrp, jbody, 0, unroll=4)
            return carry
        lax.fori_loop(0, CV, ibody, 0)
        pltpu.async_copy(ob, lap_hbm.at[pl.ds(b * nvp + vb, CV)], lsems[b])

    def ldrain(b, vb):
        pltpu.make_async_copy(lobufs[b], lap_hbm.at[pl.ds(b * nvp + vb, CV)],
                              lsems[b]).wait()

    pltpu.sync_copy(idxl_hbm.at[pl.ds(lbase * NB * lper, NB * lper)],
                    idxq0.at[pl.ds(0, NB * lper)])
    for b in range(NB):
        lgather(b, idxq0)

    def lap_body(c2, carry):
        for u in range(2):
            cc = c2 * 2 + u
            vb = wid * vpw + cc * CV
            ccn = jnp.minimum(cc + 1, nchl - 1)
            pltpu.sync_copy(
                idxl_hbm.at[pl.ds((lbase + ccn) * NB * lper, NB * lper)],
                idxqs[1 - u].at[pl.ds(0, NB * lper)])
            pltpu.sync_copy(wl_hbm.at[pl.ds((lbase + cc) * CV * 8, CV * 8)],
                            wlbuf.at[pl.ds(0, CV * 8)])
            for b in range(NB):
                lwait(b)
                if u == 0:
                    @pl.when(c2 > 0)
                    def _(b=b, vb=vb):
                        ldrain(b, vb)
                else:
                    ldrain(b, vb)
                lcompute(b, vb)
                lgather(b, idxqs[1 - u])
        return carry

    lax.fori_loop(0, nchl // 2, lap_body, 0)
    last_vb = wid * vpw + (nchl - 1) * CV
    for b in range(NB):
        lwait(b)
        ldrain(b, last_vb)


def _f2v_kernel(nf, nvp, cdim, face_hbm, idxv_hbm, wv_hbm, gve_hbm, gvn_hbm,
                idxq0, idxq1, wvbuf,
                g0, g1, g2, g3, eo0, eo1, eo2, eo3, no0, no1, no2, no3,
                sg0, sg1, sg2, sg3,
                se0, se1, se2, se3, sn0, sn1, sn2, sn3):
    wid = lax.axis_index("s") * NC + lax.axis_index("c")
    ngrp = cdim // 16
    gbufs = (g0, g1, g2, g3)
    eobufs = (eo0, eo1, eo2, eo3)
    nobufs = (no0, no1, no2, no3)
    gsems = (sg0, sg1, sg2, sg3)
    esems = (se0, se1, se2, se3)
    nsems = (sn0, sn1, sn2, sn3)
    idxqs = (idxq0, idxq1)

    vpw = nvp // NW
    nch = vpw // CV
    cbase = wid * nch
    nper = CV * 6

    def gather(b, idxq):
        pltpu.async_copy(face_hbm.at[idxq.at[pl.ds(b * nper, nper)]],
                         gbufs[b], gsems[b])

    def gwait(b):
        pltpu.make_async_copy(face_hbm.at[idxq0.at[pl.ds(0, nper)]],
                              gbufs[b], gsems[b]).wait()

    def compute(b, vb):
        gb, eo, no = gbufs[b], eobufs[b], nobufs[b]
        def ibody(i, carry, gb=gb, eo=eo, no=no):
            wvv = wvbuf[pl.ds(i * 8, 16)]
            w = [wvv[t] for t in range(6)]
            rows = [i * 6 + t for t in range(6)]
            def jbody(j, carry2, w=w, rows=rows, gb=gb, eo=eo, no=no, i=i):
                col = j * 16
                eo[i, pl.ds(col, 16)] = _wsum(gb, rows, col, w)
                no[i, pl.ds(col, 16)] = _wsum(gb, rows, cdim + col, w)
                return carry2
            lax.fori_loop(0, ngrp, jbody, 0, unroll=4)
            return carry
        lax.fori_loop(0, CV, ibody, 0)
        pltpu.async_copy(eo, gve_hbm.at[pl.ds(b * nvp + vb, CV)], esems[b])
        pltpu.async_copy(no, gvn_hbm.at[pl.ds(b * nvp + vb, CV)], nsems[b])

    def drain(b, vb):
        pltpu.make_async_copy(eobufs[b], gve_hbm.at[pl.ds(b * nvp + vb, CV)],
                              esems[b]).wait()
        pltpu.make_async_copy(nobufs[b], gvn_hbm.at[pl.ds(b * nvp + vb, CV)],
                              nsems[b]).wait()

    pltpu.sync_copy(idxv_hbm.at[pl.ds(cbase * NB * nper, NB * nper)], idxq0)
    for b in range(NB):
        gather(b, idxq0)

    def body(c2, carry):
        for u in range(2):
            cc = c2 * 2 + u
            vb = wid * vpw + cc * CV
            ccn = jnp.minimum(cc + 1, nch - 1)
            pltpu.sync_copy(
                idxv_hbm.at[pl.ds((cbase + ccn) * NB * nper, NB * nper)],
                idxqs[1 - u])
            pltpu.sync_copy(wv_hbm.at[pl.ds((cbase + cc) * CV * 8, CV * 8)],
                            wvbuf.at[pl.ds(0, CV * 8)])
            for b in range(NB):
                gwait(b)
                if u == 0:
                    @pl.when(c2 > 0)
                    def _(b=b, vb=vb):
                        drain(b, vb)
                else:
                    drain(b, vb)
                compute(b, vb)
                gather(b, idxqs[1 - u])
        return carry

    lax.fori_loop(0, nch // 2, body, 0)
    last_vb = wid * vpw + (nch - 1) * CV
    for b in range(NB):
        gwait(b)
        drain(b, last_vb)


def _matmul_kernel(x_ref, lap_ref, gve_ref, gvn_ref, w_ref, b_ref, o_ref):
    acc = jnp.dot(x_ref[...], w_ref[0], preferred_element_type=jnp.float32)
    acc += jnp.dot(lap_ref[...], w_ref[1], preferred_element_type=jnp.float32)
    acc += jnp.dot(gve_ref[...], w_ref[2], preferred_element_type=jnp.float32)
    acc += jnp.dot(gvn_ref[...], w_ref[3], preferred_element_type=jnp.float32)
    o_ref[...] = acc + b_ref[...]


def kernel(input, Gi, Gv, Li, Lv, F2Vi, F2Vv, NS, EW, coeffs, bias):
    Bsz, C, nv = input.shape
    nf = NS.shape[0]
    c_out = coeffs.shape[0]
    gran = NW * CV * 2
    nvp = ((nv + gran - 1) // gran) * gran
    f32 = jnp.float32

    xpad = jnp.pad(input.transpose(0, 2, 1), ((0, 0), (0, nvp - nv), (0, 0)))
    x4 = xpad.reshape(Bsz * nvp, C)

    bshift_v = (jnp.arange(Bsz, dtype=jnp.int32) * nvp)[:, None, None]
    bshift_f = (jnp.arange(Bsz, dtype=jnp.int32) * nf)[:, None, None]

    idxf1 = Gi[1].reshape(3, nf, 3).transpose(1, 0, 2).reshape(nf, 9)
    idxf = (idxf1[None] + bshift_v).reshape(Bsz, nf // CF, CF * 9)
    idxf = idxf.transpose(1, 0, 2).reshape(-1)

    gvr = Gv.reshape(3, nf, 3).transpose(1, 0, 2)
    we = jnp.pad((gvr * EW[:, :, None]).reshape(nf, 9), ((0, 0), (0, 7)))
    wn = jnp.pad((gvr * NS[:, :, None]).reshape(nf, 9), ((0, 0), (0, 7)))
    wen = jnp.concatenate([we, wn], axis=1).reshape(-1)

    idxl1 = jnp.pad(Li[1].reshape(nv, 7), ((0, nvp - nv), (0, 0)))
    idxl = (idxl1[None] + bshift_v).reshape(Bsz, nvp // CV, CV * 7)
    idxl = idxl.transpose(1, 0, 2).reshape(-1)
    wl = jnp.pad(Lv.reshape(nv, 7), ((0, nvp - nv), (0, 1))).reshape(-1)

    idxv1 = jnp.pad(F2Vi[1].reshape(nv, 6), ((0, nvp - nv), (0, 0)))
    idxv = (idxv1[None] + bshift_f).reshape(Bsz, nvp // CV, CV * 6)
    idxv = idxv.transpose(1, 0, 2).reshape(-1)
    wv = jnp.pad(F2Vv.reshape(nv, 6), ((0, nvp - nv), (0, 2))).reshape(-1)

    mesh = plsc.VectorSubcoreMesh(core_axis_name="c", subcore_axis_name="s",
                                  num_cores=NC, num_subcores=NSUB)

    face_lap = pl.kernel(
        functools.partial(_face_lap_kernel, nf, nvp, C),
        out_type=[
            jax.ShapeDtypeStruct((Bsz * nf, 2 * C), f32),
            jax.ShapeDtypeStruct((Bsz * nvp, C), f32),
        ],
        mesh=mesh,
        scratch_types=(
            [pltpu.VMEM((NB * CF * 9,), jnp.int32)] * 2
            + [pltpu.VMEM((CF * 32,), f32)]
            + [pltpu.VMEM((CV * 8 + 8,), f32)]
            + [pltpu.VMEM((CF * 9, C), f32)] * 4
            + [pltpu.VMEM((CF, 2 * C), f32)] * 4
            + [pltpu.VMEM((CV, C), f32)] * 4
            + [pltpu.SemaphoreType.DMA] * 12
        ),
    )
    face, lap = face_lap(x4, idxf, wen, idxl, wl)

    f2v = pl.kernel(
        functools.partial(_f2v_kernel, nf, nvp, C),
        out_type=[
            jax.ShapeDtypeStruct((Bsz * nvp, C), f32),
            jax.ShapeDtypeStruct((Bsz * nvp, C), f32),
        ],
        mesh=mesh,
        scratch_types=(
            [pltpu.VMEM((NB * CV * 6,), jnp.int32)] * 2
            + [pltpu.VMEM((CV * 8 + 8,), f32)]
            + [pltpu.VMEM((CV * 6, 2 * C), f32)] * 4
            + [pltpu.VMEM((CV, C), f32)] * 8
            + [pltpu.SemaphoreType.DMA] * 12
        ),
    )
    gve, gvn = f2v(face, idxv, wv)

    m = Bsz * nvp
    bm = 512
    w4 = coeffs.transpose(2, 1, 0)
    b2 = bias.reshape(1, c_out)

    out2 = pl.pallas_call(
        _matmul_kernel,
        grid=(m // bm,),
        in_specs=[
            pl.BlockSpec((bm, C), lambda i: (i, 0)),
            pl.BlockSpec((bm, C), lambda i: (i, 0)),
            pl.BlockSpec((bm, C), lambda i: (i, 0)),
            pl.BlockSpec((bm, C), lambda i: (i, 0)),
            pl.BlockSpec((4, C, c_out), lambda i: (0, 0, 0)),
            pl.BlockSpec((1, c_out), lambda i: (0, 0)),
        ],
        out_specs=pl.BlockSpec((bm, c_out), lambda i: (i, 0)),
        out_shape=jax.ShapeDtypeStruct((m, c_out), f32),
    )(x4, lap, gve, gvn, w4, b2)

    return out2.reshape(Bsz, nvp, c_out)[:, :nv].transpose(0, 2, 1)


# speedup vs baseline: 1.4278x; 1.4278x over previous
import functools

import jax
import jax.numpy as jnp
from jax import lax
from jax.experimental import pallas as pl
from jax.experimental.pallas import tpu as pltpu
from jax.experimental.pallas import tpu_sc as plsc

NC = 2
NSUB = 16
NW = NC * NSUB
CF = 8
CV = 8


def _wsum(gref, rows, col, wts):
    acc = wts[0] * gref[rows[0], pl.ds(col, 16)]
    for t in range(1, len(wts)):
        acc = acc + wts[t] * gref[rows[t], pl.ds(col, 16)]
    return acc


def _face_lap_kernel(nf, nvp, d2, x0, x1, idxf_hbm, wen_hbm, idxl_hbm,
                     wl_hbm, face0, face1, lap0, lap1,
                     idxA, idxB, wbuf, wlbuf, g0, g1, fo0, fo1, lo0, lo1,
                     sg0, sg1, sf0, sf1, sl0, sl1):
    wid = lax.axis_index("s") * NC + lax.axis_index("c")
    ngrp = d2 // 16
    xs = (x0, x1)
    faces = (face0, face1)
    laps = (lap0, lap1)
    gbufs = (g0, g1)
    fobufs = (fo0, fo1)
    lobufs = (lo0, lo1)
    gsems = (sg0, sg1)
    fsems = (sf0, sf1)
    lsems = (sl0, sl1)
    idxqs = (idxA, idxB)

    fpw = nf // NW
    fbase = wid * fpw
    nch = fpw // CF
    nper = CF * 9

    def fgather(h):
        pltpu.async_copy(xs[h].at[idxqs[h]], gbufs[h], gsems[h])

    def fwait(h):
        pltpu.make_async_copy(xs[h].at[idxqs[h]], gbufs[h], gsems[h]).wait()

    def fcompute(h, fb):
        gb, ob = gbufs[h], fobufs[h]
        for i in range(CF):
            wev = wbuf[pl.ds(i * 32, 16)]
            wnv = wbuf[pl.ds(i * 32 + 16, 16)]
            we = [wev[t] for t in range(9)]
            wn = [wnv[t] for t in range(9)]
            rows = [i * 9 + t for t in range(9)]
            def jbody(j, carry, i=i, we=we, wn=wn, rows=rows, gb=gb, ob=ob):
                col = j * 16
                ob[i, pl.ds(col, 16)] = _wsum(gb, rows, col, we)
                ob[i, pl.ds(d2 + col, 16)] = _wsum(gb, rows, col, wn)
                return carry
            lax.fori_loop(0, ngrp, jbody, 0)
        pltpu.async_copy(ob, faces[h].at[pl.ds(fb, CF)], fsems[h])

    def fdrain(h, fb):
        pltpu.make_async_copy(fobufs[h], faces[h].at[pl.ds(fb, CF)],
                              fsems[h]).wait()

    pltpu.sync_copy(idxf_hbm.at[pl.ds(fbase * 9, nper)], idxA)
    fgather(0)

    def face_body(c, carry):
        fb = fbase + c * CF
        fbn = jnp.minimum(fb + CF, fbase + (nch - 1) * CF)
        pltpu.sync_copy(idxf_hbm.at[pl.ds(fb * 9, nper)], idxB)
        pltpu.sync_copy(wen_hbm.at[pl.ds(fb * 32, CF * 32)], wbuf)
        fgather(1)
        fwait(0)
        @pl.when(c > 0)
        def _():
            fdrain(0, fb)
        fcompute(0, fb)
        pltpu.sync_copy(idxf_hbm.at[pl.ds(fbn * 9, nper)], idxA)
        fgather(0)
        fwait(1)
        @pl.when(c > 0)
        def _():
            fdrain(1, fb)
        fcompute(1, fb)
        return carry

    lax.fori_loop(0, nch, face_body, 0)
    fwait(0)
    last_fb = fbase + (nch - 1) * CF
    fdrain(0, last_fb)
    fdrain(1, last_fb)

    vpw = nvp // NW
    vbase = wid * vpw
    nchl = vpw // CV
    lper = CV * 7

    def lgather(h):
        pltpu.async_copy(xs[h].at[idxqs[h].at[pl.ds(0, lper)]],
                         gbufs[h].at[pl.ds(0, lper)], gsems[h])

    def lwait(h):
        pltpu.make_async_copy(xs[h].at[idxqs[h].at[pl.ds(0, lper)]],
                              gbufs[h].at[pl.ds(0, lper)], gsems[h]).wait()

    def lcompute(h, vb):
        gb, ob = gbufs[h], lobufs[h]
        for i in range(CV):
            wlv = wlbuf[pl.ds(i * 8, 16)]
            wl = [wlv[t] for t in range(7)]
            rows = [i * 7 + t for t in range(7)]
            def jbody(j, carry, i=i, wl=wl, rows=rows, gb=gb, ob=ob):
                col = j * 16
                ob[i, pl.ds(col, 16)] = _wsum(gb, rows, col, wl)
                return carry
            lax.fori_loop(0, ngrp, jbody, 0)
        pltpu.async_copy(ob, laps[h].at[pl.ds(vb, CV)], lsems[h])

    def ldrain(h, vb):
        pltpu.make_async_copy(lobufs[h], laps[h].at[pl.ds(vb, CV)],
                              lsems[h]).wait()

    pltpu.sync_copy(idxl_hbm.at[pl.ds(vbase * 7, lper)],
                    idxA.at[pl.ds(0, lper)])
    lgather(0)

    def lap_body(c, carry):
        vb = vbase + c * CV
        vbn = jnp.minimum(vb + CV, vbase + (nchl - 1) * CV)
        pltpu.sync_copy(idxl_hbm.at[pl.ds(vb * 7, lper)],
                        idxB.at[pl.ds(0, lper)])
        pltpu.sync_copy(wl_hbm.at[pl.ds(vb * 8, CV * 8)],
                        wlbuf.at[pl.ds(0, CV * 8)])
        lgather(1)
        lwait(0)
        @pl.when(c > 0)
        def _():
            ldrain(0, vb)
        lcompute(0, vb)
        pltpu.sync_copy(idxl_hbm.at[pl.ds(vbn * 7, lper)],
                        idxA.at[pl.ds(0, lper)])
        lgather(0)
        lwait(1)
        @pl.when(c > 0)
        def _():
            ldrain(1, vb)
        lcompute(1, vb)
        return carry

    lax.fori_loop(0, nchl, lap_body, 0)
    lwait(0)
    last_vb = vbase + (nchl - 1) * CV
    ldrain(0, last_vb)
    ldrain(1, last_vb)


def _f2v_kernel(nf, nvp, d2, face0, face1, idxv_hbm, wv_hbm,
                gve0, gve1, gvn0, gvn1,
                idxA, idxB, wvbuf, g0, g1, eo0, eo1, no0, no1,
                sg0, sg1, se0, se1, sn0, sn1):
    wid = lax.axis_index("s") * NC + lax.axis_index("c")
    ngrp = d2 // 16
    faces = (face0, face1)
    gves = (gve0, gve1)
    gvns = (gvn0, gvn1)
    gbufs = (g0, g1)
    eobufs = (eo0, eo1)
    nobufs = (no0, no1)
    gsems = (sg0, sg1)
    esems = (se0, se1)
    nsems = (sn0, sn1)
    idxqs = (idxA, idxB)

    vpw = nvp // NW
    vbase = wid * vpw
    nch = vpw // CV
    nper = CV * 6

    def gather(h):
        pltpu.async_copy(faces[h].at[idxqs[h]], gbufs[h], gsems[h])

    def gwait(h):
        pltpu.make_async_copy(faces[h].at[idxqs[h]], gbufs[h],
                              gsems[h]).wait()

    def compute(h, vb):
        gb, eo, no = gbufs[h], eobufs[h], nobufs[h]
        for i in range(CV):
            wvv = wvbuf[pl.ds(i * 8, 16)]
            w = [wvv[t] for t in range(6)]
            rows = [i * 6 + t for t in range(6)]
            def jbody(j, carry, i=i, w=w, rows=rows, gb=gb, eo=eo, no=no):
                col = j * 16
                eo[i, pl.ds(col, 16)] = _wsum(gb, rows, col, w)
                no[i, pl.ds(col, 16)] = _wsum(gb, rows, d2 + col, w)
                return carry
            lax.fori_loop(0, ngrp, jbody, 0)
        pltpu.async_copy(eo, gves[h].at[pl.ds(vb, CV)], esems[h])
        pltpu.async_copy(no, gvns[h].at[pl.ds(vb, CV)], nsems[h])

    def drain(h, vb):
        pltpu.make_async_copy(eobufs[h], gves[h].at[pl.ds(vb, CV)],
                              esems[h]).wait()
        pltpu.make_async_copy(nobufs[h], gvns[h].at[pl.ds(vb, CV)],
                              nsems[h]).wait()

    pltpu.sync_copy(idxv_hbm.at[pl.ds(vbase * 6, nper)], idxA)
    gather(0)

    def body(c, carry):
        vb = vbase + c * CV
        vbn = jnp.minimum(vb + CV, vbase + (nch - 1) * CV)
        pltpu.sync_copy(idxv_hbm.at[pl.ds(vb * 6, nper)], idxB)
        pltpu.sync_copy(wv_hbm.at[pl.ds(vb * 8, CV * 8)],
                        wvbuf.at[pl.ds(0, CV * 8)])
        gather(1)
        gwait(0)
        @pl.when(c > 0)
        def _():
            drain(0, vb)
        compute(0, vb)
        pltpu.sync_copy(idxv_hbm.at[pl.ds(vbn * 6, nper)], idxA)
        gather(0)
        gwait(1)
        @pl.when(c > 0)
        def _():
            drain(1, vb)
        compute(1, vb)
        return carry

    lax.fori_loop(0, nch, body, 0)
    gwait(0)
    last_vb = vbase + (nch - 1) * CV
    drain(0, last_vb)
    drain(1, last_vb)


def _transpose_kernel(x_ref, o_ref):
    c = x_ref.shape[1]
    o_ref[:, :c] = x_ref[0].T
    o_ref[:, c:] = x_ref[1].T


def _matmul_kernel(cdim, x_ref, lap_ref, gve_ref, gvn_ref, w_ref, b_ref,
                   o_ref):
    for s in (slice(0, cdim), slice(cdim, 2 * cdim)):
        acc = jnp.dot(x_ref[:, s], w_ref[0], preferred_element_type=jnp.float32)
        acc += jnp.dot(lap_ref[:, s], w_ref[1],
                       preferred_element_type=jnp.float32)
        acc += jnp.dot(gve_ref[:, s], w_ref[2],
                       preferred_element_type=jnp.float32)
        acc += jnp.dot(gvn_ref[:, s], w_ref[3],
                       preferred_element_type=jnp.float32)
        o_ref[:, s] = acc + b_ref[...]


def kernel(input, Gi, Gv, Li, Lv, F2Vi, F2Vv, NS, EW, coeffs, bias):
    Bsz, C, nv = input.shape
    nf = NS.shape[0]
    c_out = coeffs.shape[0]
    d2 = 2 * C
    gran = 512
    nvp = ((nv + gran - 1) // gran) * gran
    f32 = jnp.float32
    bn = 512
    nvb = nvp // bn

    idxf = Gi[1].reshape(3, nf, 3).transpose(1, 0, 2).reshape(-1)
    gvr = Gv.reshape(3, nf, 3).transpose(1, 0, 2)
    we = jnp.pad((gvr * EW[:, :, None]).reshape(nf, 9), ((0, 0), (0, 7)))
    wn = jnp.pad((gvr * NS[:, :, None]).reshape(nf, 9), ((0, 0), (0, 7)))
    wen = jnp.concatenate([we, wn], axis=1).reshape(-1)

    idxl = jnp.pad(Li[1].reshape(nv, 7), ((0, nvp - nv), (0, 0))).reshape(-1)
    wl = jnp.pad(Lv.reshape(nv, 7), ((0, nvp - nv), (0, 1))).reshape(-1)
    idxv = jnp.pad(F2Vi[1].reshape(nv, 6), ((0, nvp - nv), (0, 0))).reshape(-1)
    wv = jnp.pad(F2Vv.reshape(nv, 6), ((0, nvp - nv), (0, 2))).reshape(-1)

    xh = []
    for h in range(Bsz // 2):
        xh.append(pl.pallas_call(
            _transpose_kernel,
            grid=(nvb,),
            in_specs=[pl.BlockSpec((2, C, bn), lambda j, h=h: (h, 0, j))],
            out_specs=pl.BlockSpec((bn, d2), lambda j: (j, 0)),
            out_shape=jax.ShapeDtypeStruct((nvp, d2), f32),
        )(input))

    mesh = plsc.VectorSubcoreMesh(core_axis_name="c", subcore_axis_name="s",
                                  num_cores=NC, num_subcores=NSUB)

    face_lap = pl.kernel(
        functools.partial(_face_lap_kernel, nf, nvp, d2),
        out_type=[
            jax.ShapeDtypeStruct((nf, 2 * d2), f32),
            jax.ShapeDtypeStruct((nf, 2 * d2), f32),
            jax.ShapeDtypeStruct((nvp, d2), f32),
            jax.ShapeDtypeStruct((nvp, d2), f32),
        ],
        mesh=mesh,
        scratch_types=(
            [pltpu.VMEM((CF * 9,), jnp.int32)] * 2
            + [pltpu.VMEM((CF * 32,), f32)]
            + [pltpu.VMEM((CV * 8 + 8,), f32)]
            + [pltpu.VMEM((CF * 9, d2), f32)] * 2
            + [pltpu.VMEM((CF, 2 * d2), f32)] * 2
            + [pltpu.VMEM((CV, d2), f32)] * 2
            + [pltpu.SemaphoreType.DMA] * 6
        ),
    )
    face_0, face_1, lap_0, lap_1 = face_lap(xh[0], xh[1], idxf, wen,
                                            idxl, wl)

    f2v = pl.kernel(
        functools.partial(_f2v_kernel, nf, nvp, d2),
        out_type=[jax.ShapeDtypeStruct((nvp, d2), f32)] * 4,
        mesh=mesh,
        scratch_types=(
            [pltpu.VMEM((CV * 6,), jnp.int32)] * 2
            + [pltpu.VMEM((CV * 8 + 8,), f32)]
            + [pltpu.VMEM((CV * 6, 2 * d2), f32)] * 2
            + [pltpu.VMEM((CV, d2), f32)] * 4
            + [pltpu.SemaphoreType.DMA] * 6
        ),
    )
    gve_0, gve_1, gvn_0, gvn_1 = f2v(face_0, face_1, idxv, wv)

    w4 = coeffs.transpose(2, 1, 0)
    b2 = bias.reshape(1, c_out)
    outs = []
    for xt, lp, ge, gn in ((xh[0], lap_0, gve_0, gvn_0),
                           (xh[1], lap_1, gve_1, gvn_1)):
        outs.append(pl.pallas_call(
            functools.partial(_matmul_kernel, C),
            grid=(nvb,),
            in_specs=[
                pl.BlockSpec((bn, d2), lambda j: (j, 0)),
                pl.BlockSpec((bn, d2), lambda j: (j, 0)),
                pl.BlockSpec((bn, d2), lambda j: (j, 0)),
                pl.BlockSpec((bn, d2), lambda j: (j, 0)),
                pl.BlockSpec((4, C, c_out), lambda j: (0, 0, 0)),
                pl.BlockSpec((1, c_out), lambda j: (0, 0)),
            ],
            out_specs=pl.BlockSpec((bn, 2 * c_out), lambda j: (j, 0)),
            out_shape=jax.ShapeDtypeStruct((nvp, 2 * c_out), f32),
        )(xt, lp, ge, gn, w4, b2))

    out = jnp.stack([outs[0][:, :c_out], outs[0][:, c_out:],
                     outs[1][:, :c_out], outs[1][:, c_out:]], axis=0)
    return out[:, :nv].transpose(0, 2, 1)


# speedup vs baseline: 1.5202x; 1.0647x over previous
import functools

import jax
import jax.numpy as jnp
from jax import lax
from jax.experimental import pallas as pl
from jax.experimental.pallas import tpu as pltpu
from jax.experimental.pallas import tpu_sc as plsc

NC = 2
NSUB = 16
NW = NC * NSUB
CF = 8
CV = 8


def _wsum(gref, rows, col, wts):
    acc = wts[0] * gref[rows[0], pl.ds(col, 16)]
    for t in range(1, len(wts)):
        acc = acc + wts[t] * gref[rows[t], pl.ds(col, 16)]
    return acc


def _face_lap_kernel(nf, nvp, d2, x0, x1, idxf_hbm, wen_hbm, idxl_hbm,
                     wl_hbm, face0, face1, lap0, lap1,
                     idxA, idxB, wbuf, wlbuf, g0, g1, fo0, fo1, lo0, lo1,
                     sg0, sg1, sf0, sf1, sl0, sl1):
    wid = lax.axis_index("s") * NC + lax.axis_index("c")
    ngrp = d2 // 16
    xs = (x0, x1)
    faces = (face0, face1)
    laps = (lap0, lap1)
    gbufs = (g0, g1)
    fobufs = (fo0, fo1)
    lobufs = (lo0, lo1)
    gsems = (sg0, sg1)
    fsems = (sf0, sf1)
    lsems = (sl0, sl1)
    idxqs = (idxA, idxB)

    fpw = nf // NW
    fbase = wid * fpw
    nch = fpw // CF
    nper = CF * 9

    def fgather(h):
        pltpu.async_copy(xs[h].at[idxqs[h]], gbufs[h], gsems[h])

    def fwait(h):
        pltpu.make_async_copy(xs[h].at[idxqs[h]], gbufs[h], gsems[h]).wait()

    def fcompute(h, fb):
        gb, ob = gbufs[h], fobufs[h]
        for i in range(CF):
            wev = wbuf[pl.ds(i * 32, 16)]
            wnv = wbuf[pl.ds(i * 32 + 16, 16)]
            we = [wev[t] for t in range(9)]
            wn = [wnv[t] for t in range(9)]
            rows = [i * 9 + t for t in range(9)]
            def jbody(j, carry, i=i, we=we, wn=wn, rows=rows, gb=gb, ob=ob):
                col = j * 16
                ob[i, pl.ds(col, 16)] = _wsum(gb, rows, col, we)
                ob[i, pl.ds(d2 + col, 16)] = _wsum(gb, rows, col, wn)
                return carry
            lax.fori_loop(0, ngrp, jbody, 0)
        pltpu.async_copy(ob, faces[h].at[pl.ds(fb, CF)], fsems[h])

    def fdrain(h, fb):
        pltpu.make_async_copy(fobufs[h], faces[h].at[pl.ds(fb, CF)],
                              fsems[h]).wait()

    pltpu.sync_copy(idxf_hbm.at[pl.ds(fbase * 9, nper)], idxA)
    fgather(0)

    def face_body(c, carry):
        fb = fbase + c * CF
        fbn = jnp.minimum(fb + CF, fbase + (nch - 1) * CF)
        pltpu.sync_copy(idxf_hbm.at[pl.ds(fb * 9, nper)], idxB)
        pltpu.sync_copy(wen_hbm.at[pl.ds(fb * 32, CF * 32)], wbuf)
        fgather(1)
        fwait(0)
        @pl.when(c > 0)
        def _():
            fdrain(0, fb)
        fcompute(0, fb)
        pltpu.sync_copy(idxf_hbm.at[pl.ds(fbn * 9, nper)], idxA)
        fgather(0)
        fwait(1)
        @pl.when(c > 0)
        def _():
            fdrain(1, fb)
        fcompute(1, fb)
        return carry

    lax.fori_loop(0, nch, face_body, 0)
    fwait(0)
    last_fb = fbase + (nch - 1) * CF
    fdrain(0, last_fb)
    fdrain(1, last_fb)

    vpw = nvp // NW
    vbase = wid * vpw
    nchl = vpw // CV
    lper = CV * 7

    def lgather(h):
        pltpu.async_copy(xs[h].at[idxqs[h].at[pl.ds(0, lper)]],
                         gbufs[h].at[pl.ds(0, lper)], gsems[h])

    def lwait(h):
        pltpu.make_async_copy(xs[h].at[idxqs[h].at[pl.ds(0, lper)]],
                              gbufs[h].at[pl.ds(0, lper)], gsems[h]).wait()

    def lcompute(h, vb):
        gb, ob = gbufs[h], lobufs[h]
        for i in range(CV):
            wlv = wlbuf[pl.ds(i * 8, 16)]
            wl = [wlv[t] for t in range(7)]
            rows = [i * 7 + t for t in range(7)]
            def jbody(j, carry, i=i, wl=wl, rows=rows, gb=gb, ob=ob):
                col = j * 16
                ob[i, pl.ds(col, 16)] = _wsum(gb, rows, col, wl)
                return carry
            lax.fori_loop(0, ngrp, jbody, 0)
        pltpu.async_copy(ob, laps[h].at[pl.ds(vb, CV)], lsems[h])

    def ldrain(h, vb):
        pltpu.make_async_copy(lobufs[h], laps[h].at[pl.ds(vb, CV)],
                              lsems[h]).wait()

    pltpu.sync_copy(idxl_hbm.at[pl.ds(vbase * 7, lper)],
                    idxA.at[pl.ds(0, lper)])
    lgather(0)

    def lap_body(c, carry):
        vb = vbase + c * CV
        vbn = jnp.minimum(vb + CV, vbase + (nchl - 1) * CV)
        pltpu.sync_copy(idxl_hbm.at[pl.ds(vb * 7, lper)],
                        idxB.at[pl.ds(0, lper)])
        pltpu.sync_copy(wl_hbm.at[pl.ds(vb * 8, CV * 8)],
                        wlbuf.at[pl.ds(0, CV * 8)])
        lgather(1)
        lwait(0)
        @pl.when(c > 0)
        def _():
            ldrain(0, vb)
        lcompute(0, vb)
        pltpu.sync_copy(idxl_hbm.at[pl.ds(vbn * 7, lper)],
                        idxA.at[pl.ds(0, lper)])
        lgather(0)
        lwait(1)
        @pl.when(c > 0)
        def _():
            ldrain(1, vb)
        lcompute(1, vb)
        return carry

    lax.fori_loop(0, nchl, lap_body, 0)
    lwait(0)
    last_vb = vbase + (nchl - 1) * CV
    ldrain(0, last_vb)
    ldrain(1, last_vb)


def _f2v_kernel(nf, nvp, d2, face0, face1, idxv_hbm, wv_hbm,
                gve0, gve1, gvn0, gvn1,
                idxA, idxB, wvbuf, g0, g1, eo0, eo1, no0, no1,
                sg0, sg1, se0, se1, sn0, sn1):
    wid = lax.axis_index("s") * NC + lax.axis_index("c")
    ngrp = d2 // 16
    faces = (face0, face1)
    gves = (gve0, gve1)
    gvns = (gvn0, gvn1)
    gbufs = (g0, g1)
    eobufs = (eo0, eo1)
    nobufs = (no0, no1)
    gsems = (sg0, sg1)
    esems = (se0, se1)
    nsems = (sn0, sn1)
    idxqs = (idxA, idxB)

    vpw = nvp // NW
    vbase = wid * vpw
    nch = vpw // CV
    nper = CV * 6

    def gather(h):
        pltpu.async_copy(faces[h].at[idxqs[h]], gbufs[h], gsems[h])

    def gwait(h):
        pltpu.make_async_copy(faces[h].at[idxqs[h]], gbufs[h],
                              gsems[h]).wait()

    def compute(h, vb):
        gb, eo, no = gbufs[h], eobufs[h], nobufs[h]
        for i in range(CV):
            wvv = wvbuf[pl.ds(i * 8, 16)]
            w = [wvv[t] for t in range(6)]
            rows = [i * 6 + t for t in range(6)]
            def jbody(j, carry, i=i, w=w, rows=rows, gb=gb, eo=eo, no=no):
                col = j * 16
                eo[i, pl.ds(col, 16)] = _wsum(gb, rows, col, w)
                no[i, pl.ds(col, 16)] = _wsum(gb, rows, d2 + col, w)
                return carry
            lax.fori_loop(0, ngrp, jbody, 0)
        pltpu.async_copy(eo, gves[h].at[pl.ds(vb, CV)], esems[h])
        pltpu.async_copy(no, gvns[h].at[pl.ds(vb, CV)], nsems[h])

    def drain(h, vb):
        pltpu.make_async_copy(eobufs[h], gves[h].at[pl.ds(vb, CV)],
                              esems[h]).wait()
        pltpu.make_async_copy(nobufs[h], gvns[h].at[pl.ds(vb, CV)],
                              nsems[h]).wait()

    pltpu.sync_copy(idxv_hbm.at[pl.ds(vbase * 6, nper)], idxA)
    gather(0)

    def body(c, carry):
        vb = vbase + c * CV
        vbn = jnp.minimum(vb + CV, vbase + (nch - 1) * CV)
        pltpu.sync_copy(idxv_hbm.at[pl.ds(vb * 6, nper)], idxB)
        pltpu.sync_copy(wv_hbm.at[pl.ds(vb * 8, CV * 8)],
                        wvbuf.at[pl.ds(0, CV * 8)])
        gather(1)
        gwait(0)
        @pl.when(c > 0)
        def _():
            drain(0, vb)
        compute(0, vb)
        pltpu.sync_copy(idxv_hbm.at[pl.ds(vbn * 6, nper)], idxA)
        gather(0)
        gwait(1)
        @pl.when(c > 0)
        def _():
            drain(1, vb)
        compute(1, vb)
        return carry

    lax.fori_loop(0, nch, body, 0)
    gwait(0)
    last_vb = vbase + (nch - 1) * CV
    drain(0, last_vb)
    drain(1, last_vb)


def _transpose_kernel(x_ref, o_ref):
    c = x_ref.shape[1]
    o_ref[:, :c] = x_ref[0].T
    o_ref[:, c:] = x_ref[1].T


def _matmul_kernel(cdim, x_ref, lap_ref, gve_ref, gvn_ref, w_ref, b_ref,
                   o_ref):
    accs = []
    for s in (slice(0, cdim), slice(cdim, 2 * cdim)):
        acc = jnp.dot(x_ref[:, s], w_ref[0], preferred_element_type=jnp.float32)
        acc += jnp.dot(lap_ref[:, s], w_ref[1],
                       preferred_element_type=jnp.float32)
        acc += jnp.dot(gve_ref[:, s], w_ref[2],
                       preferred_element_type=jnp.float32)
        acc += jnp.dot(gvn_ref[:, s], w_ref[3],
                       preferred_element_type=jnp.float32)
        accs.append(acc + b_ref[...])
    o_ref[...] = jnp.concatenate(accs, axis=1).T


def kernel(input, Gi, Gv, Li, Lv, F2Vi, F2Vv, NS, EW, coeffs, bias):
    Bsz, C, nv = input.shape
    nf = NS.shape[0]
    c_out = coeffs.shape[0]
    d2 = 2 * C
    gran = 512
    nvp = ((nv + gran - 1) // gran) * gran
    f32 = jnp.float32
    bn = 512
    nvb = nvp // bn

    idxf = Gi[1].reshape(3, nf, 3).transpose(1, 0, 2).reshape(-1)
    gvr = Gv.reshape(3, nf, 3).transpose(1, 0, 2)
    we = jnp.pad((gvr * EW[:, :, None]).reshape(nf, 9), ((0, 0), (0, 7)))
    wn = jnp.pad((gvr * NS[:, :, None]).reshape(nf, 9), ((0, 0), (0, 7)))
    wen = jnp.concatenate([we, wn], axis=1).reshape(-1)

    idxl = jnp.pad(Li[1].reshape(nv, 7), ((0, nvp - nv), (0, 0))).reshape(-1)
    wl = jnp.pad(Lv.reshape(nv, 7), ((0, nvp - nv), (0, 1))).reshape(-1)
    idxv = jnp.pad(F2Vi[1].reshape(nv, 6), ((0, nvp - nv), (0, 0))).reshape(-1)
    wv = jnp.pad(F2Vv.reshape(nv, 6), ((0, nvp - nv), (0, 2))).reshape(-1)

    xh = []
    for h in range(Bsz // 2):
        xh.append(pl.pallas_call(
            _transpose_kernel,
            grid=(nvb,),
            in_specs=[pl.BlockSpec((2, C, bn), lambda j, h=h: (h, 0, j))],
            out_specs=pl.BlockSpec((bn, d2), lambda j: (j, 0)),
            out_shape=jax.ShapeDtypeStruct((nvp, d2), f32),
        )(input))

    mesh = plsc.VectorSubcoreMesh(core_axis_name="c", subcore_axis_name="s",
                                  num_cores=NC, num_subcores=NSUB)

    face_lap = pl.kernel(
        functools.partial(_face_lap_kernel, nf, nvp, d2),
        out_type=[
            jax.ShapeDtypeStruct((nf, 2 * d2), f32),
            jax.ShapeDtypeStruct((nf, 2 * d2), f32),
            jax.ShapeDtypeStruct((nvp, d2), f32),
            jax.ShapeDtypeStruct((nvp, d2), f32),
        ],
        mesh=mesh,
        scratch_types=(
            [pltpu.VMEM((CF * 9,), jnp.int32)] * 2
            + [pltpu.VMEM((CF * 32,), f32)]
            + [pltpu.VMEM((CV * 8 + 8,), f32)]
            + [pltpu.VMEM((CF * 9, d2), f32)] * 2
            + [pltpu.VMEM((CF, 2 * d2), f32)] * 2
            + [pltpu.VMEM((CV, d2), f32)] * 2
            + [pltpu.SemaphoreType.DMA] * 6
        ),
    )
    face_0, face_1, lap_0, lap_1 = face_lap(xh[0], xh[1], idxf, wen,
                                            idxl, wl)

    f2v = pl.kernel(
        functools.partial(_f2v_kernel, nf, nvp, d2),
        out_type=[jax.ShapeDtypeStruct((nvp, d2), f32)] * 4,
        mesh=mesh,
        scratch_types=(
            [pltpu.VMEM((CV * 6,), jnp.int32)] * 2
            + [pltpu.VMEM((CV * 8 + 8,), f32)]
            + [pltpu.VMEM((CV * 6, 2 * d2), f32)] * 2
            + [pltpu.VMEM((CV, d2), f32)] * 4
            + [pltpu.SemaphoreType.DMA] * 6
        ),
    )
    gve_0, gve_1, gvn_0, gvn_1 = f2v(face_0, face_1, idxv, wv)

    w4 = coeffs.transpose(2, 1, 0)
    b2 = bias.reshape(1, c_out)
    outs = []
    for xt, lp, ge, gn in ((xh[0], lap_0, gve_0, gvn_0),
                           (xh[1], lap_1, gve_1, gvn_1)):
        outs.append(pl.pallas_call(
            functools.partial(_matmul_kernel, C),
            grid=(nvb,),
            in_specs=[
                pl.BlockSpec((bn, d2), lambda j: (j, 0)),
                pl.BlockSpec((bn, d2), lambda j: (j, 0)),
                pl.BlockSpec((bn, d2), lambda j: (j, 0)),
                pl.BlockSpec((bn, d2), lambda j: (j, 0)),
                pl.BlockSpec((4, C, c_out), lambda j: (0, 0, 0)),
                pl.BlockSpec((1, c_out), lambda j: (0, 0)),
            ],
            out_specs=pl.BlockSpec((2 * c_out, bn), lambda j: (0, j)),
            out_shape=jax.ShapeDtypeStruct((2 * c_out, nvp), f32),
        )(xt, lp, ge, gn, w4, b2))

    out = jnp.concatenate(outs, axis=0).reshape(Bsz, c_out, nvp)
    return out[:, :, :nv]


# speedup vs baseline: 1.5514x; 1.0205x over previous
import functools

import jax
import jax.numpy as jnp
from jax import lax
from jax.experimental import pallas as pl
from jax.experimental.pallas import tpu as pltpu
from jax.experimental.pallas import tpu_sc as plsc

NC = 2
NSUB = 16
NW = NC * NSUB
CF = 8
CV = 8


def _wsum(gref, rows, col, wts):
    acc = wts[0] * gref[rows[0], pl.ds(col, 16)]
    for t in range(1, len(wts)):
        acc = acc + wts[t] * gref[rows[t], pl.ds(col, 16)]
    return acc


def _face_lap_kernel(nf, nvp, d2, x0, x1, idxf_hbm, wen_hbm, idxl_hbm,
                     wl_hbm, face0, face1, lap0, lap1,
                     idxA, idxB, wbA, wbB, wlA, wlB,
                     g0, g1, fo0, fo1, lo0, lo1,
                     sg0, sg1, sf0, sf1, sl0, sl1):
    wid = lax.axis_index("s") * NC + lax.axis_index("c")
    ngrp = d2 // 16
    xs = (x0, x1)
    faces = (face0, face1)
    laps = (lap0, lap1)
    gbufs = (g0, g1)
    fobufs = (fo0, fo1)
    lobufs = (lo0, lo1)
    gsems = (sg0, sg1)
    fsems = (sf0, sf1)
    lsems = (sl0, sl1)
    idxqs = (idxA, idxB)
    wqs = (wbA, wbB)
    wlqs = (wlA, wlB)

    fpw = nf // NW
    fbase = wid * fpw
    nch = fpw // CF
    nper = CF * 9

    def fgather(h, p):
        pltpu.async_copy(xs[h].at[idxqs[p]], gbufs[h], gsems[h])

    def fwait(h):
        pltpu.make_async_copy(xs[h].at[idxqs[0]], gbufs[h], gsems[h]).wait()

    def fcompute(h, fb, wbuf):
        gb, ob = gbufs[h], fobufs[h]
        for i in range(CF):
            wev = wbuf[pl.ds(i * 32, 16)]
            wnv = wbuf[pl.ds(i * 32 + 16, 16)]
            we = [wev[t] for t in range(9)]
            wn = [wnv[t] for t in range(9)]
            rows = [i * 9 + t for t in range(9)]
            def jbody(j, carry, i=i, we=we, wn=wn, rows=rows, gb=gb, ob=ob):
                col = j * 16
                ob[i, pl.ds(col, 16)] = _wsum(gb, rows, col, we)
                ob[i, pl.ds(d2 + col, 16)] = _wsum(gb, rows, col, wn)
                return carry
            lax.fori_loop(0, ngrp, jbody, 0)
        pltpu.async_copy(ob, faces[h].at[pl.ds(fb, CF)], fsems[h])

    def fdrain(h, fb):
        pltpu.make_async_copy(fobufs[h], faces[h].at[pl.ds(fb, CF)],
                              fsems[h]).wait()

    pltpu.sync_copy(idxf_hbm.at[pl.ds(fbase * 9, nper)], idxA)
    pltpu.sync_copy(wen_hbm.at[pl.ds(fbase * 32, CF * 32)], wbA)
    fgather(0, 0)
    fgather(1, 0)

    def face_body(c2, carry):
        for u in range(2):
            cc = c2 * 2 + u
            fb = fbase + cc * CF
            fbn = jnp.minimum(fb + CF, fbase + (nch - 1) * CF)
            pltpu.sync_copy(idxf_hbm.at[pl.ds(fbn * 9, nper)],
                            idxqs[1 - u])
            pltpu.sync_copy(wen_hbm.at[pl.ds(fbn * 32, CF * 32)],
                            wqs[1 - u])
            for h in range(2):
                fwait(h)
                if u == 0:
                    @pl.when(c2 > 0)
                    def _(h=h, fb=fb):
                        fdrain(h, fb)
                else:
                    fdrain(h, fb)
                fcompute(h, fb, wqs[u])
                fgather(h, 1 - u)
        return carry

    lax.fori_loop(0, nch // 2, face_body, 0)
    last_fb = fbase + (nch - 1) * CF
    for h in range(2):
        fwait(h)
        fdrain(h, last_fb)

    vpw = nvp // NW
    vbase = wid * vpw
    nchl = vpw // CV
    lper = CV * 7

    def lgather(h, p):
        pltpu.async_copy(xs[h].at[idxqs[p].at[pl.ds(0, lper)]],
                         gbufs[h].at[pl.ds(0, lper)], gsems[h])

    def lwait(h):
        pltpu.make_async_copy(xs[h].at[idxqs[0].at[pl.ds(0, lper)]],
                              gbufs[h].at[pl.ds(0, lper)], gsems[h]).wait()

    def lcompute(h, vb, wlbuf):
        gb, ob = gbufs[h], lobufs[h]
        for i in range(CV):
            wlv = wlbuf[pl.ds(i * 8, 16)]
            wl = [wlv[t] for t in range(7)]
            rows = [i * 7 + t for t in range(7)]
            def jbody(j, carry, i=i, wl=wl, rows=rows, gb=gb, ob=ob):
                col = j * 16
                ob[i, pl.ds(col, 16)] = _wsum(gb, rows, col, wl)
                return carry
            lax.fori_loop(0, ngrp, jbody, 0)
        pltpu.async_copy(ob, laps[h].at[pl.ds(vb, CV)], lsems[h])

    def ldrain(h, vb):
        pltpu.make_async_copy(lobufs[h], laps[h].at[pl.ds(vb, CV)],
                              lsems[h]).wait()

    pltpu.sync_copy(idxl_hbm.at[pl.ds(vbase * 7, lper)],
                    idxA.at[pl.ds(0, lper)])
    pltpu.sync_copy(wl_hbm.at[pl.ds(vbase * 8, CV * 8)],
                    wlA.at[pl.ds(0, CV * 8)])
    lgather(0, 0)
    lgather(1, 0)

    def lap_body(c2, carry):
        for u in range(2):
            cc = c2 * 2 + u
            vb = vbase + cc * CV
            vbn = jnp.minimum(vb + CV, vbase + (nchl - 1) * CV)
            pltpu.sync_copy(idxl_hbm.at[pl.ds(vbn * 7, lper)],
                            idxqs[1 - u].at[pl.ds(0, lper)])
            pltpu.sync_copy(wl_hbm.at[pl.ds(vbn * 8, CV * 8)],
                            wlqs[1 - u].at[pl.ds(0, CV * 8)])
            for h in range(2):
                lwait(h)
                if u == 0:
                    @pl.when(c2 > 0)
                    def _(h=h, vb=vb):
                        ldrain(h, vb)
                else:
                    ldrain(h, vb)
                lcompute(h, vb, wlqs[u])
                lgather(h, 1 - u)
        return carry

    lax.fori_loop(0, nchl // 2, lap_body, 0)
    last_vb = vbase + (nchl - 1) * CV
    for h in range(2):
        lwait(h)
        ldrain(h, last_vb)


def _f2v_kernel(nf, nvp, d2, face0, face1, idxv_hbm, wv_hbm,
                gve0, gve1, gvn0, gvn1,
                idxA, idxB, wvA, wvB, g0, g1, eo0, eo1, no0, no1,
                sg0, sg1, se0, se1, sn0, sn1):
    wid = lax.axis_index("s") * NC + lax.axis_index("c")
    ngrp = d2 // 16
    faces = (face0, face1)
    gves = (gve0, gve1)
    gvns = (gvn0, gvn1)
    gbufs = (g0, g1)
    eobufs = (eo0, eo1)
    nobufs = (no0, no1)
    gsems = (sg0, sg1)
    esems = (se0, se1)
    nsems = (sn0, sn1)
    idxqs = (idxA, idxB)
    wqs = (wvA, wvB)

    vpw = nvp // NW
    vbase = wid * vpw
    nch = vpw // CV
    nper = CV * 6

    def gather(h, p):
        pltpu.async_copy(faces[h].at[idxqs[p]], gbufs[h], gsems[h])

    def gwait(h):
        pltpu.make_async_copy(faces[h].at[idxqs[0]], gbufs[h],
                              gsems[h]).wait()

    def compute(h, vb, wvbuf):
        gb, eo, no = gbufs[h], eobufs[h], nobufs[h]
        for i in range(CV):
            wvv = wvbuf[pl.ds(i * 8, 16)]
            w = [wvv[t] for t in range(6)]
            rows = [i * 6 + t for t in range(6)]
            def jbody(j, carry, i=i, w=w, rows=rows, gb=gb, eo=eo, no=no):
                col = j * 16
                eo[i, pl.ds(col, 16)] = _wsum(gb, rows, col, w)
                no[i, pl.ds(col, 16)] = _wsum(gb, rows, d2 + col, w)
                return carry
            lax.fori_loop(0, ngrp, jbody, 0)
        pltpu.async_copy(eo, gves[h].at[pl.ds(vb, CV)], esems[h])
        pltpu.async_copy(no, gvns[h].at[pl.ds(vb, CV)], nsems[h])

    def drain(h, vb):
        pltpu.make_async_copy(eobufs[h], gves[h].at[pl.ds(vb, CV)],
                              esems[h]).wait()
        pltpu.make_async_copy(nobufs[h], gvns[h].at[pl.ds(vb, CV)],
                              nsems[h]).wait()

    pltpu.sync_copy(idxv_hbm.at[pl.ds(vbase * 6, nper)], idxA)
    pltpu.sync_copy(wv_hbm.at[pl.ds(vbase * 8, CV * 8)],
                    wvA.at[pl.ds(0, CV * 8)])
    gather(0, 0)
    gather(1, 0)

    def body(c2, carry):
        for u in range(2):
            cc = c2 * 2 + u
            vb = vbase + cc * CV
            vbn = jnp.minimum(vb + CV, vbase + (nch - 1) * CV)
            pltpu.sync_copy(idxv_hbm.at[pl.ds(vbn * 6, nper)],
                            idxqs[1 - u])
            pltpu.sync_copy(wv_hbm.at[pl.ds(vbn * 8, CV * 8)],
                            wqs[1 - u].at[pl.ds(0, CV * 8)])
            for h in range(2):
                gwait(h)
                if u == 0:
                    @pl.when(c2 > 0)
                    def _(h=h, vb=vb):
                        drain(h, vb)
                else:
                    drain(h, vb)
                compute(h, vb, wqs[u])
                gather(h, 1 - u)
        return carry

    lax.fori_loop(0, nch // 2, body, 0)
    last_vb = vbase + (nch - 1) * CV
    for h in range(2):
        gwait(h)
        drain(h, last_vb)


def _transpose_kernel(x_ref, o_ref):
    c = x_ref.shape[1]
    o_ref[:, :c] = x_ref[0].T
    o_ref[:, c:] = x_ref[1].T


def _matmul_kernel(cdim, x_ref, lap_ref, gve_ref, gvn_ref, w_ref, b_ref,
                   o_ref):
    accs = []
    for s in (slice(0, cdim), slice(cdim, 2 * cdim)):
        acc = jnp.dot(x_ref[:, s], w_ref[0], preferred_element_type=jnp.float32)
        acc += jnp.dot(lap_ref[:, s], w_ref[1],
                       preferred_element_type=jnp.float32)
        acc += jnp.dot(gve_ref[:, s], w_ref[2],
                       preferred_element_type=jnp.float32)
        acc += jnp.dot(gvn_ref[:, s], w_ref[3],
                       preferred_element_type=jnp.float32)
        accs.append(acc + b_ref[...])
    o_ref[...] = jnp.concatenate(accs, axis=1).T


def kernel(input, Gi, Gv, Li, Lv, F2Vi, F2Vv, NS, EW, coeffs, bias):
    Bsz, C, nv = input.shape
    nf = NS.shape[0]
    c_out = coeffs.shape[0]
    d2 = 2 * C
    gran = 512
    nvp = ((nv + gran - 1) // gran) * gran
    f32 = jnp.float32
    bn = 512
    nvb = nvp // bn

    idxf = Gi[1].reshape(3, nf, 3).transpose(1, 0, 2).reshape(-1)
    gvr = Gv.reshape(3, nf, 3).transpose(1, 0, 2)
    we = jnp.pad((gvr * EW[:, :, None]).reshape(nf, 9), ((0, 0), (0, 7)))
    wn = jnp.pad((gvr * NS[:, :, None]).reshape(nf, 9), ((0, 0), (0, 7)))
    wen = jnp.concatenate([we, wn], axis=1).reshape(-1)

    idxl = jnp.pad(Li[1].reshape(nv, 7), ((0, nvp - nv), (0, 0))).reshape(-1)
    wl = jnp.pad(Lv.reshape(nv, 7), ((0, nvp - nv), (0, 1))).reshape(-1)
    idxv = jnp.pad(F2Vi[1].reshape(nv, 6), ((0, nvp - nv), (0, 0))).reshape(-1)
    wv = jnp.pad(F2Vv.reshape(nv, 6), ((0, nvp - nv), (0, 2))).reshape(-1)

    xh = []
    for h in range(Bsz // 2):
        xh.append(pl.pallas_call(
            _transpose_kernel,
            grid=(nvb,),
            in_specs=[pl.BlockSpec((2, C, bn), lambda j, h=h: (h, 0, j))],
            out_specs=pl.BlockSpec((bn, d2), lambda j: (j, 0)),
            out_shape=jax.ShapeDtypeStruct((nvp, d2), f32),
        )(input))

    mesh = plsc.VectorSubcoreMesh(core_axis_name="c", subcore_axis_name="s",
                                  num_cores=NC, num_subcores=NSUB)

    face_lap = pl.kernel(
        functools.partial(_face_lap_kernel, nf, nvp, d2),
        out_type=[
            jax.ShapeDtypeStruct((nf, 2 * d2), f32),
            jax.ShapeDtypeStruct((nf, 2 * d2), f32),
            jax.ShapeDtypeStruct((nvp, d2), f32),
            jax.ShapeDtypeStruct((nvp, d2), f32),
        ],
        mesh=mesh,
        scratch_types=(
            [pltpu.VMEM((CF * 9,), jnp.int32)] * 2
            + [pltpu.VMEM((CF * 32,), f32)] * 2
            + [pltpu.VMEM((CV * 8 + 8,), f32)] * 2
            + [pltpu.VMEM((CF * 9, d2), f32)] * 2
            + [pltpu.VMEM((CF, 2 * d2), f32)] * 2
            + [pltpu.VMEM((CV, d2), f32)] * 2
            + [pltpu.SemaphoreType.DMA] * 6
        ),
    )
    face_0, face_1, lap_0, lap_1 = face_lap(xh[0], xh[1], idxf, wen,
                                            idxl, wl)

    f2v = pl.kernel(
        functools.partial(_f2v_kernel, nf, nvp, d2),
        out_type=[jax.ShapeDtypeStruct((nvp, d2), f32)] * 4,
        mesh=mesh,
        scratch_types=(
            [pltpu.VMEM((CV * 6,), jnp.int32)] * 2
            + [pltpu.VMEM((CV * 8 + 8,), f32)] * 2
            + [pltpu.VMEM((CV * 6, 2 * d2), f32)] * 2
            + [pltpu.VMEM((CV, d2), f32)] * 4
            + [pltpu.SemaphoreType.DMA] * 6
        ),
    )
    gve_0, gve_1, gvn_0, gvn_1 = f2v(face_0, face_1, idxv, wv)

    w4 = coeffs.transpose(2, 1, 0)
    b2 = bias.reshape(1, c_out)
    outs = []
    for xt, lp, ge, gn in ((xh[0], lap_0, gve_0, gvn_0),
                           (xh[1], lap_1, gve_1, gvn_1)):
        outs.append(pl.pallas_call(
            functools.partial(_matmul_kernel, C),
            grid=(nvb,),
            in_specs=[
                pl.BlockSpec((bn, d2), lambda j: (j, 0)),
                pl.BlockSpec((bn, d2), lambda j: (j, 0)),
                pl.BlockSpec((bn, d2), lambda j: (j, 0)),
                pl.BlockSpec((bn, d2), lambda j: (j, 0)),
                pl.BlockSpec((4, C, c_out), lambda j: (0, 0, 0)),
                pl.BlockSpec((1, c_out), lambda j: (0, 0)),
            ],
            out_specs=pl.BlockSpec((2 * c_out, bn), lambda j: (0, j)),
            out_shape=jax.ShapeDtypeStruct((2 * c_out, nvp), f32),
        )(xt, lp, ge, gn, w4, b2))

    out = jnp.concatenate(outs, axis=0).reshape(Bsz, c_out, nvp)
    return out[:, :, :nv]


# speedup vs baseline: 1.5980x; 1.0300x over previous
import functools

import jax
import jax.numpy as jnp
from jax import lax
from jax.experimental import pallas as pl
from jax.experimental.pallas import tpu as pltpu
from jax.experimental.pallas import tpu_sc as plsc

NC = 2
NSUB = 16
NW = NC * NSUB
CF = 8
CV = 8


def _wsum(gref, rows, col, wts):
    acc = wts[0] * gref[rows[0], pl.ds(col, 16)]
    for t in range(1, len(wts)):
        acc = acc + wts[t] * gref[rows[t], pl.ds(col, 16)]
    return acc


def _face_lap_kernel(nf, nvp, d2, x0, x1, idxf_hbm, wen_hbm, idxl_hbm,
                     wl_hbm, face0, face1, lap0, lap1,
                     idxA, idxB, wbA, wbB, wlA, wlB,
                     g0, g1, fo0, fo1, lo0, lo1,
                     sg0, sg1, sf0, sf1, sl0, sl1):
    wid = lax.axis_index("s") * NC + lax.axis_index("c")
    ngrp = d2 // 16
    xs = (x0, x1)
    faces = (face0, face1)
    laps = (lap0, lap1)
    gbufs = (g0, g1)
    fobufs = (fo0, fo1)
    lobufs = (lo0, lo1)
    gsems = (sg0, sg1)
    fsems = (sf0, sf1)
    lsems = (sl0, sl1)
    idxqs = (idxA, idxB)
    wqs = (wbA, wbB)
    wlqs = (wlA, wlB)

    fpw = nf // NW
    fbase = wid * fpw
    nch = fpw // CF
    nper = CF * 9

    def fgather(h, p):
        pltpu.async_copy(xs[h].at[idxqs[p]], gbufs[h], gsems[h])

    def fwait(h):
        pltpu.make_async_copy(xs[h].at[idxqs[0]], gbufs[h], gsems[h]).wait()

    def fcompute(h, fb, wbuf):
        gb, ob = gbufs[h], fobufs[h]
        for i in range(CF):
            wev = wbuf[pl.ds(i * 32, 16)]
            wnv = wbuf[pl.ds(i * 32 + 16, 16)]
            we = [wev[t] for t in range(9)]
            wn = [wnv[t] for t in range(9)]
            rows = [i * 9 + t for t in range(9)]
            def jbody(j, carry, i=i, we=we, wn=wn, rows=rows, gb=gb, ob=ob):
                col = j * 16
                ob[i, pl.ds(col, 16)] = _wsum(gb, rows, col, we)
                ob[i, pl.ds(d2 + col, 16)] = _wsum(gb, rows, col, wn)
                return carry
            lax.fori_loop(0, ngrp, jbody, 0)
        pltpu.async_copy(ob, faces[h].at[pl.ds(fb, CF)], fsems[h])

    def fdrain(h, fb):
        pltpu.make_async_copy(fobufs[h], faces[h].at[pl.ds(fb, CF)],
                              fsems[h]).wait()

    pltpu.sync_copy(idxf_hbm.at[pl.ds(fbase * 9, nper)], idxA)
    pltpu.sync_copy(wen_hbm.at[pl.ds(fbase * 32, CF * 32)], wbA)
    fgather(0, 0)
    fgather(1, 0)

    def face_body(c2, carry):
        for u in range(2):
            cc = c2 * 2 + u
            fb = fbase + cc * CF
            fbn = jnp.minimum(fb + CF, fbase + (nch - 1) * CF)
            pltpu.sync_copy(idxf_hbm.at[pl.ds(fbn * 9, nper)],
                            idxqs[1 - u])
            pltpu.sync_copy(wen_hbm.at[pl.ds(fbn * 32, CF * 32)],
                            wqs[1 - u])
            for h in range(2):
                fwait(h)
                if u == 0:
                    @pl.when(c2 > 0)
                    def _(h=h, fb=fb):
                        fdrain(h, fb)
                else:
                    fdrain(h, fb)
                fcompute(h, fb, wqs[u])
                fgather(h, 1 - u)
        return carry

    lax.fori_loop(0, nch // 2, face_body, 0)
    last_fb = fbase + (nch - 1) * CF
    for h in range(2):
        fwait(h)
        fdrain(h, last_fb)

    vpw = nvp // NW
    vbase = wid * vpw
    nchl = vpw // CV
    lper = CV * 7

    def lgather(h, p):
        pltpu.async_copy(xs[h].at[idxqs[p].at[pl.ds(0, lper)]],
                         gbufs[h].at[pl.ds(0, lper)], gsems[h])

    def lwait(h):
        pltpu.make_async_copy(xs[h].at[idxqs[0].at[pl.ds(0, lper)]],
                              gbufs[h].at[pl.ds(0, lper)], gsems[h]).wait()

    def lcompute(h, vb, wlbuf):
        gb, ob = gbufs[h], lobufs[h]
        for i in range(CV):
            wlv = wlbuf[pl.ds(i * 8, 16)]
            wl = [wlv[t] for t in range(7)]
            rows = [i * 7 + t for t in range(7)]
            def jbody(j, carry, i=i, wl=wl, rows=rows, gb=gb, ob=ob):
                col = j * 16
                ob[i, pl.ds(col, 16)] = _wsum(gb, rows, col, wl)
                return carry
            lax.fori_loop(0, ngrp, jbody, 0)
        pltpu.async_copy(ob, laps[h].at[pl.ds(vb, CV)], lsems[h])

    def ldrain(h, vb):
        pltpu.make_async_copy(lobufs[h], laps[h].at[pl.ds(vb, CV)],
                              lsems[h]).wait()

    pltpu.sync_copy(idxl_hbm.at[pl.ds(vbase * 7, lper)],
                    idxA.at[pl.ds(0, lper)])
    pltpu.sync_copy(wl_hbm.at[pl.ds(vbase * 8, CV * 8)],
                    wlA.at[pl.ds(0, CV * 8)])
    lgather(0, 0)
    lgather(1, 0)

    def lap_body(c2, carry):
        for u in range(2):
            cc = c2 * 2 + u
            vb = vbase + cc * CV
            vbn = jnp.minimum(vb + CV, vbase + (nchl - 1) * CV)
            pltpu.sync_copy(idxl_hbm.at[pl.ds(vbn * 7, lper)],
                            idxqs[1 - u].at[pl.ds(0, lper)])
            pltpu.sync_copy(wl_hbm.at[pl.ds(vbn * 8, CV * 8)],
                            wlqs[1 - u].at[pl.ds(0, CV * 8)])
            for h in range(2):
                lwait(h)
                if u == 0:
                    @pl.when(c2 > 0)
                    def _(h=h, vb=vb):
                        ldrain(h, vb)
                else:
                    ldrain(h, vb)
                lcompute(h, vb, wlqs[u])
                lgather(h, 1 - u)
        return carry

    lax.fori_loop(0, nchl // 2, lap_body, 0)
    last_vb = vbase + (nchl - 1) * CV
    for h in range(2):
        lwait(h)
        ldrain(h, last_vb)


def _f2v_kernel(nf, nvp, d2, face0, face1, idxv_hbm, wv_hbm,
                gve0, gve1, gvn0, gvn1,
                idxA, idxB, wvA, wvB, g0, g1, eo0, eo1, no0, no1,
                sg0, sg1, se0, se1, sn0, sn1):
    wid = lax.axis_index("s") * NC + lax.axis_index("c")
    ngrp = d2 // 16
    faces = (face0, face1)
    gves = (gve0, gve1)
    gvns = (gvn0, gvn1)
    gbufs = (g0, g1)
    eobufs = (eo0, eo1)
    nobufs = (no0, no1)
    gsems = (sg0, sg1)
    esems = (se0, se1)
    nsems = (sn0, sn1)
    idxqs = (idxA, idxB)
    wqs = (wvA, wvB)

    vpw = nvp // NW
    vbase = wid * vpw
    nch = vpw // CV
    nper = CV * 6

    def gather(h, p):
        pltpu.async_copy(faces[h].at[idxqs[p]], gbufs[h], gsems[h])

    def gwait(h):
        pltpu.make_async_copy(faces[h].at[idxqs[0]], gbufs[h],
                              gsems[h]).wait()

    def compute(h, vb, wvbuf):
        gb, eo, no = gbufs[h], eobufs[h], nobufs[h]
        for i in range(CV):
            wvv = wvbuf[pl.ds(i * 8, 16)]
            w = [wvv[t] for t in range(6)]
            rows = [i * 6 + t for t in range(6)]
            def jbody(j, carry, i=i, w=w, rows=rows, gb=gb, eo=eo, no=no):
                col = j * 16
                eo[i, pl.ds(col, 16)] = _wsum(gb, rows, col, w)
                no[i, pl.ds(col, 16)] = _wsum(gb, rows, d2 + col, w)
                return carry
            lax.fori_loop(0, ngrp, jbody, 0)
        pltpu.async_copy(eo, gves[h].at[pl.ds(vb, CV)], esems[h])
        pltpu.async_copy(no, gvns[h].at[pl.ds(vb, CV)], nsems[h])

    def drain(h, vb):
        pltpu.make_async_copy(eobufs[h], gves[h].at[pl.ds(vb, CV)],
                              esems[h]).wait()
        pltpu.make_async_copy(nobufs[h], gvns[h].at[pl.ds(vb, CV)],
                              nsems[h]).wait()

    pltpu.sync_copy(idxv_hbm.at[pl.ds(vbase * 6, nper)], idxA)
    pltpu.sync_copy(wv_hbm.at[pl.ds(vbase * 8, CV * 8)],
                    wvA.at[pl.ds(0, CV * 8)])
    gather(0, 0)
    gather(1, 0)

    def body(c2, carry):
        for u in range(2):
            cc = c2 * 2 + u
            vb = vbase + cc * CV
            vbn = jnp.minimum(vb + CV, vbase + (nch - 1) * CV)
            pltpu.sync_copy(idxv_hbm.at[pl.ds(vbn * 6, nper)],
                            idxqs[1 - u])
            pltpu.sync_copy(wv_hbm.at[pl.ds(vbn * 8, CV * 8)],
                            wqs[1 - u].at[pl.ds(0, CV * 8)])
            for h in range(2):
                gwait(h)
                if u == 0:
                    @pl.when(c2 > 0)
                    def _(h=h, vb=vb):
                        drain(h, vb)
                else:
                    drain(h, vb)
                compute(h, vb, wqs[u])
                gather(h, 1 - u)
        return carry

    lax.fori_loop(0, nch // 2, body, 0)
    last_vb = vbase + (nch - 1) * CV
    for h in range(2):
        gwait(h)
        drain(h, last_vb)


def _transpose_kernel(x0_ref, x1_ref, o0_ref, o1_ref):
    c = x0_ref.shape[1]
    o0_ref[:, :c] = x0_ref[0].T
    o0_ref[:, c:] = x0_ref[1].T
    o1_ref[:, :c] = x1_ref[0].T
    o1_ref[:, c:] = x1_ref[1].T


def _matmul_kernel(cdim, x0_ref, lap0_ref, gve0_ref, gvn0_ref,
                   x1_ref, lap1_ref, gve1_ref, gvn1_ref, w_ref, b_ref,
                   o0_ref, o1_ref):
    for feats, o_ref in (((x0_ref, lap0_ref, gve0_ref, gvn0_ref), o0_ref),
                         ((x1_ref, lap1_ref, gve1_ref, gvn1_ref), o1_ref)):
        accs = []
        for s in (slice(0, cdim), slice(cdim, 2 * cdim)):
            acc = jnp.dot(feats[0][:, s], w_ref[0],
                          preferred_element_type=jnp.float32)
            for k in range(1, 4):
                acc += jnp.dot(feats[k][:, s], w_ref[k],
                               preferred_element_type=jnp.float32)
            accs.append(acc + b_ref[...])
        o_ref[...] = jnp.concatenate(accs, axis=1).T


def kernel(input, Gi, Gv, Li, Lv, F2Vi, F2Vv, NS, EW, coeffs, bias):
    Bsz, C, nv = input.shape
    nf = NS.shape[0]
    c_out = coeffs.shape[0]
    d2 = 2 * C
    gran = 512
    nvp = ((nv + gran - 1) // gran) * gran
    f32 = jnp.float32
    bn = 512
    nvb = nvp // bn

    idxf = Gi[1].reshape(3, nf, 3).transpose(1, 0, 2).reshape(-1)
    gvr = Gv.reshape(3, nf, 3).transpose(1, 0, 2)
    we = jnp.pad((gvr * EW[:, :, None]).reshape(nf, 9), ((0, 0), (0, 7)))
    wn = jnp.pad((gvr * NS[:, :, None]).reshape(nf, 9), ((0, 0), (0, 7)))
    wen = jnp.concatenate([we, wn], axis=1).reshape(-1)

    idxl = jnp.pad(Li[1].reshape(nv, 7), ((0, nvp - nv), (0, 0))).reshape(-1)
    wl = jnp.pad(Lv.reshape(nv, 7), ((0, nvp - nv), (0, 1))).reshape(-1)
    idxv = jnp.pad(F2Vi[1].reshape(nv, 6), ((0, nvp - nv), (0, 0))).reshape(-1)
    wv = jnp.pad(F2Vv.reshape(nv, 6), ((0, nvp - nv), (0, 2))).reshape(-1)

    xh = pl.pallas_call(
        _transpose_kernel,
        grid=(nvb,),
        in_specs=[pl.BlockSpec((2, C, bn), lambda j: (0, 0, j)),
                  pl.BlockSpec((2, C, bn), lambda j: (1, 0, j))],
        out_specs=[pl.BlockSpec((bn, d2), lambda j: (j, 0))] * 2,
        out_shape=[jax.ShapeDtypeStruct((nvp, d2), f32)] * 2,
    )(input, input)

    mesh = plsc.VectorSubcoreMesh(core_axis_name="c", subcore_axis_name="s",
                                  num_cores=NC, num_subcores=NSUB)

    face_lap = pl.kernel(
        functools.partial(_face_lap_kernel, nf, nvp, d2),
        out_type=[
            jax.ShapeDtypeStruct((nf, 2 * d2), f32),
            jax.ShapeDtypeStruct((nf, 2 * d2), f32),
            jax.ShapeDtypeStruct((nvp, d2), f32),
            jax.ShapeDtypeStruct((nvp, d2), f32),
        ],
        mesh=mesh,
        scratch_types=(
            [pltpu.VMEM((CF * 9,), jnp.int32)] * 2
            + [pltpu.VMEM((CF * 32,), f32)] * 2
            + [pltpu.VMEM((CV * 8 + 8,), f32)] * 2
            + [pltpu.VMEM((CF * 9, d2), f32)] * 2
            + [pltpu.VMEM((CF, 2 * d2), f32)] * 2
            + [pltpu.VMEM((CV, d2), f32)] * 2
            + [pltpu.SemaphoreType.DMA] * 6
        ),
    )
    face_0, face_1, lap_0, lap_1 = face_lap(xh[0], xh[1], idxf, wen,
                                            idxl, wl)

    f2v = pl.kernel(
        functools.partial(_f2v_kernel, nf, nvp, d2),
        out_type=[jax.ShapeDtypeStruct((nvp, d2), f32)] * 4,
        mesh=mesh,
        scratch_types=(
            [pltpu.VMEM((CV * 6,), jnp.int32)] * 2
            + [pltpu.VMEM((CV * 8 + 8,), f32)] * 2
            + [pltpu.VMEM((CV * 6, 2 * d2), f32)] * 2
            + [pltpu.VMEM((CV, d2), f32)] * 4
            + [pltpu.SemaphoreType.DMA] * 6
        ),
    )
    gve_0, gve_1, gvn_0, gvn_1 = f2v(face_0, face_1, idxv, wv)

    w4 = coeffs.transpose(2, 1, 0)
    b2 = bias.reshape(1, c_out)
    outs = pl.pallas_call(
        functools.partial(_matmul_kernel, C),
        grid=(nvb,),
        in_specs=(
            [pl.BlockSpec((bn, d2), lambda j: (j, 0))] * 8
            + [pl.BlockSpec((4, C, c_out), lambda j: (0, 0, 0)),
               pl.BlockSpec((1, c_out), lambda j: (0, 0))]
        ),
        out_specs=[pl.BlockSpec((2 * c_out, bn), lambda j: (0, j))] * 2,
        out_shape=[jax.ShapeDtypeStruct((2 * c_out, nvp), f32)] * 2,
    )(xh[0], lap_0, gve_0, gvn_0, xh[1], lap_1, gve_1, gvn_1, w4, b2)

    out = jnp.concatenate(outs, axis=0).reshape(Bsz, c_out, nvp)
    return out[:, :, :nv]
